# Initial kernel scaffold; baseline (speedup 1.0000x reference)
import jax
import jax.numpy as jnp
from jax.experimental import pallas as pl


def kernel(x_paper, x_author, edge_index_pa, edge_index_ap, edge_index_pp, W_fc1_paper, b_fc1_paper, W_fc1_author, b_fc1_author, W_fc2, b_fc2, attn1, attn2, sem_W1, sem_b1, sem_W2):
    raise NotImplementedError("write your pallas kernel here")


# speedup vs baseline: 7.5265x; 7.5265x over previous
import functools

import jax
import jax.numpy as jnp
from jax import lax
from jax.experimental import pallas as pl
from jax.experimental.pallas import tpu as pltpu
from jax.experimental.pallas import tpu_sc as plsc

N = 5000
NPAD = 5120
E = 160000
D = 64
AUG = 80
HOP = 5
NC = 2
NS = 16
NW = NC * NS
K = 128
NCH = 40
EPT = K * NCH
EPAD = EPT * NW
RPT = NPAD // NS


def _elu(x):
    return jnp.where(x > 0, x, jnp.exp(jnp.minimum(x, 0.0)) - 1.0)


def _exp_leaky(x):
    return jnp.exp(jnp.where(x >= 0, x, 0.2 * x))


def _rowmat(a_row, x):
    return lax.dot_general(a_row, x, (((1,), (1,)), ((), ())),
                           preferred_element_type=jnp.float32)


def _colmat(x, a_row):
    return lax.dot_general(x, a_row, (((1,), (1,)), ((), ())),
                           preferred_element_type=jnp.float32)


def _prologue_body(xp_ref, xa_ref, w1p_ref, b1p_ref, w1a_ref, b1a_ref,
                   a1_ref, a2_ref,
                   xtp_ref, xta_ref, haugp_ref, hauga_ref, x1_ref, w2_ref,
                   h10_ref):
    xtp = jnp.maximum(jnp.dot(xp_ref[...], w1p_ref[...],
                              preferred_element_type=jnp.float32)
                      + b1p_ref[...], 0.0)
    xta = jnp.maximum(jnp.dot(xa_ref[...], w1a_ref[...],
                              preferred_element_type=jnp.float32)
                      + b1a_ref[...], 0.0)
    xtp_ref[...] = xtp
    xta_ref[...] = xta

    for ref, xt in ((haugp_ref, xtp), (hauga_ref, xta)):
        ref[...] = jnp.zeros((NPAD, AUG), jnp.float32)
        ref[0:N, 0:D] = xt
        ref[0:N, D:D + 1] = jnp.ones((N, 1), jnp.float32)

    x1_ref[...] = jnp.zeros((HOP, 3, NPAD), jnp.float32)
    xts = (xtp, xta, xtp)
    for i in range(HOP):
        for et in range(3):
            x = xts[et]
            a1 = a1_ref[i, et]
            a2 = a2_ref[i, et]
            x1_ref[i, et:et + 1, 0:N] = _rowmat(a1, x)
            w2_ref[i, :, et:et + 1] = _exp_leaky(_colmat(x, a1)
                                                 + _colmat(x, a2))

    h10_ref[...] = jnp.zeros((3, NPAD), jnp.float32)
    hts = (xta, xtp, xtp)
    for et in range(3):
        h10_ref[et:et + 1, 0:N] = _rowmat(a2_ref[0, et], hts[et])


_prologue = pl.pallas_call(
    _prologue_body,
    out_shape=(
        jax.ShapeDtypeStruct((N, D), jnp.float32),
        jax.ShapeDtypeStruct((N, D), jnp.float32),
        jax.ShapeDtypeStruct((NPAD, AUG), jnp.float32),
        jax.ShapeDtypeStruct((NPAD, AUG), jnp.float32),
        jax.ShapeDtypeStruct((HOP, 3, NPAD), jnp.float32),
        jax.ShapeDtypeStruct((HOP, N, 3), jnp.float32),
        jax.ShapeDtypeStruct((3, NPAD), jnp.float32),
    ),
)


def _sc_body(s_pa, t_pa, s_ap, t_ap, s_pp, t_pp, x1h, h1h, haugp, hauga,
             zrows,
             o_pa, o_ap, o_pp,
             acc0, acc1, acc2, sref, tref, x1tab, h1tab, rows_g, sem):
    cid = lax.axis_index("c")
    sid = lax.axis_index("s")
    wid = sid * NC + cid
    accs = (acc0, acc1, acc2)
    outs = (o_pa, o_ap, o_pp)
    svs = (s_pa, s_ap, s_pp)
    tvs = (t_pa, t_ap, t_pp)
    htabs = (hauga, haugp, haugp)
    row0 = sid * RPT

    for a in accs:
        pltpu.sync_copy(zrows, a.at[pl.ds(row0, RPT)])
    plsc.subcore_barrier()

    pltpu.sync_copy(x1h, x1tab)
    pltpu.sync_copy(h1h, h1tab)

    for et in range(3):
        pltpu.sync_copy(svs[et].at[wid], sref)
        pltpu.sync_copy(tvs[et].at[wid], tref)
        htab = htabs[et]
        acc = accs[et]
        et16 = jnp.full((16,), et, jnp.int32)

        def chunk(j, carry):
            pltpu.async_copy(htab.at[tref.at[j]], rows_g, sem).wait()

            def group(g, c2):
                off = pl.multiple_of(g * 16, 16)
                s16 = sref[j, pl.ds(off, 16)]
                t16 = tref[j, pl.ds(off, 16)]
                x1v = plsc.load_gather(x1tab, [et16, s16])
                h1v = plsc.load_gather(h1tab, [et16, t16])
                w1v = _exp_leaky(x1v + h1v)
                ebase = g * 16
                for k in range(16):
                    wb = jnp.take_along_axis(
                        w1v, jnp.full((16,), k, jnp.int32), axis=0)
                    e = ebase + k
                    for c in range(AUG // 16):
                        rows_g[e, pl.ds(c * 16, 16)] = (
                            rows_g[e, pl.ds(c * 16, 16)] * wb)
                return c2

            lax.fori_loop(0, K // 16, group, 0)
            pltpu.sync_copy(rows_g, acc.at[sref.at[j]], add=True)
            return carry

        lax.fori_loop(0, NCH, chunk, 0)

    plsc.subcore_barrier()
    for et in range(3):
        pltpu.sync_copy(accs[et].at[pl.ds(row0, RPT)],
                        outs[et].at[cid, pl.ds(row0, RPT)])


@functools.cache
def _get_sc_edge():
  return pl.kernel(
    _sc_body,
    out_type=tuple(jax.ShapeDtypeStruct((NC, NPAD, AUG), jnp.float32)
                   for _ in range(3)),
    mesh=plsc.VectorSubcoreMesh(core_axis_name="c", subcore_axis_name="s",
                                num_cores=NC, num_subcores=NS),
    compiler_params=pltpu.CompilerParams(use_tc_tiling_on_sc=False,
                                         needs_layout_passes=False),
    scratch_types=(
        pltpu.VMEM_SHARED((NPAD, AUG), jnp.float32),
        pltpu.VMEM_SHARED((NPAD, AUG), jnp.float32),
        pltpu.VMEM_SHARED((NPAD, AUG), jnp.float32),
        pltpu.VMEM((NCH, K), jnp.int32),
        pltpu.VMEM((NCH, K), jnp.int32),
        pltpu.VMEM((3, NPAD), jnp.float32),
        pltpu.VMEM((3, NPAD), jnp.float32),
        pltpu.VMEM((K, AUG), jnp.float32),
        pltpu.SemaphoreType.DMA,
    ),
  )


def _sem_from_acc(acc_ref, xt, w2col):
    a = acc_ref[0, 0:N, 0:D] + acc_ref[1, 0:N, 0:D]
    d = acc_ref[0, 0:N, D:D + 1] + acc_ref[1, 0:N, D:D + 1]
    return (a + w2col * xt) / (d + w2col)


def _paper_update(z_pa, z_pp, sw1_ref, sb1_ref, sw2_ref):
    s1 = jnp.dot(jnp.tanh(jnp.dot(z_pa, sw1_ref[...],
                                  preferred_element_type=jnp.float32)
                          + sb1_ref[...]),
                 sw2_ref[...], preferred_element_type=jnp.float32)
    s2 = jnp.dot(jnp.tanh(jnp.dot(z_pp, sw1_ref[...],
                                  preferred_element_type=jnp.float32)
                          + sb1_ref[...]),
                 sw2_ref[...], preferred_element_type=jnp.float32)
    m1 = jnp.mean(s1)
    m2 = jnp.mean(s2)
    mx = jnp.maximum(m1, m2)
    e1 = jnp.exp(m1 - mx)
    e2 = jnp.exp(m2 - mx)
    tot = e1 + e2
    return _elu((e1 / tot) * z_pa + (e2 / tot) * z_pp)


def _combine_body(accpa_ref, accap_ref, accpp_ref, xtp_ref, xta_ref, w2_ref,
                  sw1_ref, sb1_ref, sw2_ref, a2n_ref,
                  haugp_ref, hauga_ref, h1n_ref):
    xtp = xtp_ref[...]
    xta = xta_ref[...]
    z_pa = _sem_from_acc(accpa_ref, xtp, w2_ref[:, 0:1])
    z_ap = _sem_from_acc(accap_ref, xta, w2_ref[:, 1:2])
    z_pp = _sem_from_acc(accpp_ref, xtp, w2_ref[:, 2:3])
    hp = _paper_update(z_pa, z_pp, sw1_ref, sb1_ref, sw2_ref)
    ha = _elu(z_ap)
    for ref, h in ((haugp_ref, hp), (hauga_ref, ha)):
        ref[...] = jnp.zeros((NPAD, AUG), jnp.float32)
        ref[0:N, 0:D] = h
        ref[0:N, D:D + 1] = jnp.ones((N, 1), jnp.float32)
    h1n_ref[...] = jnp.zeros((3, NPAD), jnp.float32)
    hts = (ha, hp, hp)
    for et in range(3):
        h1n_ref[et:et + 1, 0:N] = _rowmat(a2n_ref[et], hts[et])


_combine = pl.pallas_call(
    _combine_body,
    out_shape=(
        jax.ShapeDtypeStruct((NPAD, AUG), jnp.float32),
        jax.ShapeDtypeStruct((NPAD, AUG), jnp.float32),
        jax.ShapeDtypeStruct((3, NPAD), jnp.float32),
    ),
)


def _final_body(accpa_ref, accpp_ref, xtp_ref, w2_ref,
                sw1_ref, sb1_ref, sw2_ref, wfc2_ref, bfc2_ref, out_ref):
    xtp = xtp_ref[...]
    z_pa = _sem_from_acc(accpa_ref, xtp, w2_ref[:, 0:1])
    z_pp = _sem_from_acc(accpp_ref, xtp, w2_ref[:, 2:3])
    hp = _paper_update(z_pa, z_pp, sw1_ref, sb1_ref, sw2_ref)
    out_ref[...] = jnp.dot(hp, wfc2_ref[...],
                           preferred_element_type=jnp.float32) + bfc2_ref[...]


_final = pl.pallas_call(
    _final_body,
    out_shape=jax.ShapeDtypeStruct((N, 16), jnp.float32),
)


def kernel(x_paper, x_author, edge_index_pa, edge_index_ap, edge_index_pp,
           W_fc1_paper, b_fc1_paper, W_fc1_author, b_fc1_author,
           W_fc2, b_fc2, attn1, attn2, sem_W1, sem_b1, sem_W2):
    idx = []
    for ei in (edge_index_pa, edge_index_ap, edge_index_pp):
        ei = ei.astype(jnp.int32)
        s = jnp.concatenate([ei[0], jnp.full((EPAD - E,), N, jnp.int32)])
        t = jnp.concatenate([ei[1], jnp.zeros((EPAD - E,), jnp.int32)])
        idx.append((s.reshape(NW, NCH, K), t.reshape(NW, NCH, K)))
    zrows = jnp.zeros((RPT, AUG), jnp.float32)

    a1r = attn1.reshape(HOP, 3, 1, D)
    a2r = attn2.reshape(HOP, 3, 1, D)
    xtp, xta, haugp, hauga, x1_all, w2_all, h1 = _prologue(
        x_paper, x_author, W_fc1_paper, b_fc1_paper.reshape(1, D),
        W_fc1_author, b_fc1_author.reshape(1, D), a1r, a2r)

    out = None
    for i in range(HOP):
        acc_pa, acc_ap, acc_pp = _get_sc_edge()(
            idx[0][0], idx[0][1], idx[1][0], idx[1][1], idx[2][0], idx[2][1],
            x1_all[i], h1, haugp, hauga, zrows)
        sw1 = sem_W1[i, 0]
        sb1 = sem_b1[i, 0].reshape(1, -1)
        sw2 = sem_W2[i, 0].reshape(-1, 1)
        if i < HOP - 1:
            haugp, hauga, h1 = _combine(
                acc_pa, acc_ap, acc_pp, xtp, xta, w2_all[i],
                sw1, sb1, sw2, a2r[i + 1])
        else:
            out = _final(acc_pa, acc_pp, xtp, w2_all[i],
                         sw1, sb1, sw2, W_fc2, b_fc2.reshape(1, -1))
    return out


# speedup vs baseline: 11.3723x; 1.5110x over previous
import functools

import jax
import jax.numpy as jnp
from jax import lax
from jax.experimental import pallas as pl
from jax.experimental.pallas import tpu as pltpu
from jax.experimental.pallas import tpu_sc as plsc

N = 5000
NPAD = 5120
E = 160000
D = 64
AUG = 80
HOP = 5
NC = 2
NS = 16
NW = NC * NS
K = 64
NCH = 80
EPT = K * NCH
EPAD = EPT * NW
RPT = NPAD // NS


def _elu(x):
    return jnp.where(x > 0, x, jnp.exp(jnp.minimum(x, 0.0)) - 1.0)


def _exp_leaky(x):
    return jnp.exp(jnp.where(x >= 0, x, 0.2 * x))


def _rowmat(a_row, x):
    return lax.dot_general(a_row, x, (((1,), (1,)), ((), ())),
                           preferred_element_type=jnp.float32)


def _colmat(x, a_row):
    return lax.dot_general(x, a_row, (((1,), (1,)), ((), ())),
                           preferred_element_type=jnp.float32)


def _prologue_body(xp_ref, xa_ref, w1p_ref, b1p_ref, w1a_ref, b1a_ref,
                   a1_ref, a2_ref,
                   xtp_ref, xta_ref, haugp_ref, hauga_ref, x1_ref, w2_ref,
                   h10_ref):
    xtp = jnp.maximum(jnp.dot(xp_ref[...], w1p_ref[...],
                              preferred_element_type=jnp.float32)
                      + b1p_ref[...], 0.0)
    xta = jnp.maximum(jnp.dot(xa_ref[...], w1a_ref[...],
                              preferred_element_type=jnp.float32)
                      + b1a_ref[...], 0.0)
    xtp_ref[...] = xtp
    xta_ref[...] = xta

    for ref, xt in ((haugp_ref, xtp), (hauga_ref, xta)):
        ref[...] = jnp.zeros((NPAD, AUG), jnp.float32)
        ref[0:N, 0:D] = xt
        ref[0:N, D:D + 1] = jnp.ones((N, 1), jnp.float32)

    x1_ref[...] = jnp.zeros((HOP, 3, NPAD), jnp.float32)
    xts = (xtp, xta, xtp)
    for i in range(HOP):
        for et in range(3):
            x = xts[et]
            a1 = a1_ref[i, et]
            a2 = a2_ref[i, et]
            x1_ref[i, et:et + 1, 0:N] = _rowmat(a1, x)
            w2_ref[i, :, et:et + 1] = _exp_leaky(_colmat(x, a1)
                                                 + _colmat(x, a2))

    h10_ref[...] = jnp.zeros((3, NPAD), jnp.float32)
    hts = (xta, xtp, xtp)
    for et in range(3):
        h10_ref[et:et + 1, 0:N] = _rowmat(a2_ref[0, et], hts[et])


_prologue = pl.pallas_call(
    _prologue_body,
    out_shape=(
        jax.ShapeDtypeStruct((N, D), jnp.float32),
        jax.ShapeDtypeStruct((N, D), jnp.float32),
        jax.ShapeDtypeStruct((NPAD, AUG), jnp.float32),
        jax.ShapeDtypeStruct((NPAD, AUG), jnp.float32),
        jax.ShapeDtypeStruct((HOP, 3, NPAD), jnp.float32),
        jax.ShapeDtypeStruct((HOP, N, 3), jnp.float32),
        jax.ShapeDtypeStruct((3, NPAD), jnp.float32),
    ),
)


def _sc_body(s_pa, t_pa, s_ap, t_ap, s_pp, t_pp, x1h, h1h, haugp, hauga,
             zrows,
             o_pa, o_ap, o_pp,
             acc0, acc1, acc2, sref, tref, x1tab, h1tab, rows_a, rows_b,
             sem_a, sem_b):
    cid = lax.axis_index("c")
    sid = lax.axis_index("s")
    wid = sid * NC + cid
    accs = (acc0, acc1, acc2)
    outs = (o_pa, o_ap, o_pp)
    svs = (s_pa, s_ap, s_pp)
    tvs = (t_pa, t_ap, t_pp)
    htabs = (hauga, haugp, haugp)
    row0 = sid * RPT

    for a in accs:
        pltpu.sync_copy(zrows, a.at[pl.ds(row0, RPT)])
    plsc.subcore_barrier()

    pltpu.sync_copy(x1h, x1tab)
    pltpu.sync_copy(h1h, h1tab)

    for et in range(3):
        pltpu.sync_copy(svs[et].at[wid], sref)
        pltpu.sync_copy(tvs[et].at[wid], tref)
        htab = htabs[et]
        acc = accs[et]
        et16 = jnp.full((16,), et, jnp.int32)

        bufs = (rows_a, rows_b)
        sems = (sem_a, sem_b)
        pltpu.async_copy(htab.at[tref.at[0]], rows_a, sem_a)
        pltpu.async_copy(htab.at[tref.at[1]], rows_b, sem_b)

        def chunk2(i2, carry):
            for b in range(2):
                j = i2 * 2 + b
                rows = bufs[b]
                pltpu.make_async_copy(htab.at[tref.at[j]], rows,
                                      sems[b]).wait()

                def group(g, c2):
                    off = pl.multiple_of(g * 16, 16)
                    s16 = sref[j, pl.ds(off, 16)]
                    t16 = tref[j, pl.ds(off, 16)]
                    x1v = plsc.load_gather(x1tab, [et16, s16])
                    h1v = plsc.load_gather(h1tab, [et16, t16])
                    w1v = _exp_leaky(x1v + h1v)
                    ebase = g * 16
                    for k in range(16):
                        wb = jnp.take_along_axis(
                            w1v, jnp.full((16,), k, jnp.int32), axis=0)
                        e = ebase + k
                        for c in range(AUG // 16):
                            rows[e, pl.ds(c * 16, 16)] = (
                                rows[e, pl.ds(c * 16, 16)] * wb)
                    return c2

                lax.fori_loop(0, K // 16, group, 0)
                pltpu.sync_copy(rows, acc.at[sref.at[j]], add=True)

                @pl.when(j + 2 < NCH)
                def _():
                    pltpu.async_copy(htab.at[tref.at[j + 2]], rows, sems[b])
            return carry

        lax.fori_loop(0, NCH // 2, chunk2, 0)

    plsc.subcore_barrier()
    for et in range(3):
        pltpu.sync_copy(accs[et].at[pl.ds(row0, RPT)],
                        outs[et].at[cid, pl.ds(row0, RPT)])


@functools.cache
def _get_sc_edge():
  return pl.kernel(
    _sc_body,
    out_type=tuple(jax.ShapeDtypeStruct((NC, NPAD, AUG), jnp.float32)
                   for _ in range(3)),
    mesh=plsc.VectorSubcoreMesh(core_axis_name="c", subcore_axis_name="s",
                                num_cores=NC, num_subcores=NS),
    compiler_params=pltpu.CompilerParams(use_tc_tiling_on_sc=False,
                                         needs_layout_passes=False),
    scratch_types=(
        pltpu.VMEM_SHARED((NPAD, AUG), jnp.float32),
        pltpu.VMEM_SHARED((NPAD, AUG), jnp.float32),
        pltpu.VMEM_SHARED((NPAD, AUG), jnp.float32),
        pltpu.VMEM((NCH, K), jnp.int32),
        pltpu.VMEM((NCH, K), jnp.int32),
        pltpu.VMEM((3, NPAD), jnp.float32),
        pltpu.VMEM((3, NPAD), jnp.float32),
        pltpu.VMEM((K, AUG), jnp.float32),
        pltpu.VMEM((K, AUG), jnp.float32),
        pltpu.SemaphoreType.DMA,
        pltpu.SemaphoreType.DMA,
    ),
  )


def _sem_from_acc(acc_ref, xt, w2col):
    a = acc_ref[0, 0:N, 0:D] + acc_ref[1, 0:N, 0:D]
    d = acc_ref[0, 0:N, D:D + 1] + acc_ref[1, 0:N, D:D + 1]
    return (a + w2col * xt) / (d + w2col)


def _paper_update(z_pa, z_pp, sw1_ref, sb1_ref, sw2_ref):
    s1 = jnp.dot(jnp.tanh(jnp.dot(z_pa, sw1_ref[...],
                                  preferred_element_type=jnp.float32)
                          + sb1_ref[...]),
                 sw2_ref[...], preferred_element_type=jnp.float32)
    s2 = jnp.dot(jnp.tanh(jnp.dot(z_pp, sw1_ref[...],
                                  preferred_element_type=jnp.float32)
                          + sb1_ref[...]),
                 sw2_ref[...], preferred_element_type=jnp.float32)
    m1 = jnp.mean(s1)
    m2 = jnp.mean(s2)
    mx = jnp.maximum(m1, m2)
    e1 = jnp.exp(m1 - mx)
    e2 = jnp.exp(m2 - mx)
    tot = e1 + e2
    return _elu((e1 / tot) * z_pa + (e2 / tot) * z_pp)


def _combine_body(accpa_ref, accap_ref, accpp_ref, xtp_ref, xta_ref, w2_ref,
                  sw1_ref, sb1_ref, sw2_ref, a2n_ref,
                  haugp_ref, hauga_ref, h1n_ref):
    xtp = xtp_ref[...]
    xta = xta_ref[...]
    z_pa = _sem_from_acc(accpa_ref, xtp, w2_ref[:, 0:1])
    z_ap = _sem_from_acc(accap_ref, xta, w2_ref[:, 1:2])
    z_pp = _sem_from_acc(accpp_ref, xtp, w2_ref[:, 2:3])
    hp = _paper_update(z_pa, z_pp, sw1_ref, sb1_ref, sw2_ref)
    ha = _elu(z_ap)
    for ref, h in ((haugp_ref, hp), (hauga_ref, ha)):
        ref[...] = jnp.zeros((NPAD, AUG), jnp.float32)
        ref[0:N, 0:D] = h
        ref[0:N, D:D + 1] = jnp.ones((N, 1), jnp.float32)
    h1n_ref[...] = jnp.zeros((3, NPAD), jnp.float32)
    hts = (ha, hp, hp)
    for et in range(3):
        h1n_ref[et:et + 1, 0:N] = _rowmat(a2n_ref[et], hts[et])


_combine = pl.pallas_call(
    _combine_body,
    out_shape=(
        jax.ShapeDtypeStruct((NPAD, AUG), jnp.float32),
        jax.ShapeDtypeStruct((NPAD, AUG), jnp.float32),
        jax.ShapeDtypeStruct((3, NPAD), jnp.float32),
    ),
)


def _final_body(accpa_ref, accpp_ref, xtp_ref, w2_ref,
                sw1_ref, sb1_ref, sw2_ref, wfc2_ref, bfc2_ref, out_ref):
    xtp = xtp_ref[...]
    z_pa = _sem_from_acc(accpa_ref, xtp, w2_ref[:, 0:1])
    z_pp = _sem_from_acc(accpp_ref, xtp, w2_ref[:, 2:3])
    hp = _paper_update(z_pa, z_pp, sw1_ref, sb1_ref, sw2_ref)
    out_ref[...] = jnp.dot(hp, wfc2_ref[...],
                           preferred_element_type=jnp.float32) + bfc2_ref[...]


_final = pl.pallas_call(
    _final_body,
    out_shape=jax.ShapeDtypeStruct((N, 16), jnp.float32),
)


def kernel(x_paper, x_author, edge_index_pa, edge_index_ap, edge_index_pp,
           W_fc1_paper, b_fc1_paper, W_fc1_author, b_fc1_author,
           W_fc2, b_fc2, attn1, attn2, sem_W1, sem_b1, sem_W2):
    idx = []
    for ei in (edge_index_pa, edge_index_ap, edge_index_pp):
        ei = ei.astype(jnp.int32)
        s = jnp.concatenate([ei[0], jnp.full((EPAD - E,), N, jnp.int32)])
        t = jnp.concatenate([ei[1], jnp.zeros((EPAD - E,), jnp.int32)])
        idx.append((s.reshape(NW, NCH, K), t.reshape(NW, NCH, K)))
    zrows = jnp.zeros((RPT, AUG), jnp.float32)

    a1r = attn1.reshape(HOP, 3, 1, D)
    a2r = attn2.reshape(HOP, 3, 1, D)
    xtp, xta, haugp, hauga, x1_all, w2_all, h1 = _prologue(
        x_paper, x_author, W_fc1_paper, b_fc1_paper.reshape(1, D),
        W_fc1_author, b_fc1_author.reshape(1, D), a1r, a2r)

    out = None
    for i in range(HOP):
        acc_pa, acc_ap, acc_pp = _get_sc_edge()(
            idx[0][0], idx[0][1], idx[1][0], idx[1][1], idx[2][0], idx[2][1],
            x1_all[i], h1, haugp, hauga, zrows)
        sw1 = sem_W1[i, 0]
        sb1 = sem_b1[i, 0].reshape(1, -1)
        sw2 = sem_W2[i, 0].reshape(-1, 1)
        if i < HOP - 1:
            haugp, hauga, h1 = _combine(
                acc_pa, acc_ap, acc_pp, xtp, xta, w2_all[i],
                sw1, sb1, sw2, a2r[i + 1])
        else:
            out = _final(acc_pa, acc_pp, xtp, w2_all[i],
                         sw1, sb1, sw2, W_fc2, b_fc2.reshape(1, -1))
    return out


# speedup vs baseline: 11.6945x; 1.0283x over previous
import functools

import jax
import jax.numpy as jnp
from jax import lax
from jax.experimental import pallas as pl
from jax.experimental.pallas import tpu as pltpu
from jax.experimental.pallas import tpu_sc as plsc

N = 5000
NPAD = 5120
E = 160000
D = 64
AUG = 80
HOP = 5
NC = 2
NS = 16
NW = NC * NS
K = 64
NCH = 80
EPT = K * NCH
EPAD = EPT * NW
RPT = NPAD // NS


def _elu(x):
    return jnp.where(x > 0, x, jnp.exp(jnp.minimum(x, 0.0)) - 1.0)


def _exp_leaky(x):
    return jnp.exp(jnp.where(x >= 0, x, 0.2 * x))


def _rowmat(a_row, x):
    return lax.dot_general(a_row, x, (((1,), (1,)), ((), ())),
                           preferred_element_type=jnp.float32)


def _colmat(x, a_row):
    return lax.dot_general(x, a_row, (((1,), (1,)), ((), ())),
                           preferred_element_type=jnp.float32)


def _prologue_body(xp_ref, xa_ref, w1p_ref, b1p_ref, w1a_ref, b1a_ref,
                   a1_ref, a2_ref,
                   xtp_ref, xta_ref, haugp_ref, hauga_ref, x1_ref, w2_ref,
                   h10_ref):
    xtp = jnp.maximum(jnp.dot(xp_ref[...], w1p_ref[...],
                              preferred_element_type=jnp.float32)
                      + b1p_ref[...], 0.0)
    xta = jnp.maximum(jnp.dot(xa_ref[...], w1a_ref[...],
                              preferred_element_type=jnp.float32)
                      + b1a_ref[...], 0.0)
    xtp_ref[...] = xtp
    xta_ref[...] = xta

    for ref, xt in ((haugp_ref, xtp), (hauga_ref, xta)):
        ref[...] = jnp.zeros((NPAD, AUG), jnp.float32)
        ref[0:N, 0:D] = xt
        ref[0:N, D:D + 1] = jnp.ones((N, 1), jnp.float32)

    x1_ref[...] = jnp.zeros((HOP, 3, NPAD), jnp.float32)
    xts = (xtp, xta, xtp)
    for i in range(HOP):
        for et in range(3):
            x = xts[et]
            a1 = a1_ref[i, et]
            a2 = a2_ref[i, et]
            x1_ref[i, et:et + 1, 0:N] = _rowmat(a1, x)
            w2_ref[i, :, et:et + 1] = _exp_leaky(_colmat(x, a1)
                                                 + _colmat(x, a2))

    h10_ref[...] = jnp.zeros((3, NPAD), jnp.float32)
    hts = (xta, xtp, xtp)
    for et in range(3):
        h10_ref[et:et + 1, 0:N] = _rowmat(a2_ref[0, et], hts[et])


_prologue = pl.pallas_call(
    _prologue_body,
    out_shape=(
        jax.ShapeDtypeStruct((N, D), jnp.float32),
        jax.ShapeDtypeStruct((N, D), jnp.float32),
        jax.ShapeDtypeStruct((NPAD, AUG), jnp.float32),
        jax.ShapeDtypeStruct((NPAD, AUG), jnp.float32),
        jax.ShapeDtypeStruct((HOP, 3, NPAD), jnp.float32),
        jax.ShapeDtypeStruct((HOP, N, 3), jnp.float32),
        jax.ShapeDtypeStruct((3, NPAD), jnp.float32),
    ),
)


def _sc_body(s_pa, t_pa, s_ap, t_ap, s_pp, t_pp, x1h, h1h, haugp, hauga,
             zrows,
             o_pa, o_ap, o_pp,
             acc0, acc1, acc2, sref, tref, x1tab, h1tab,
             buf0, buf1, buf2, buf3, g0, g1, g2, g3, s0, s1, s2, s3):
    cid = lax.axis_index("c")
    sid = lax.axis_index("s")
    wid = sid * NC + cid
    accs = (acc0, acc1, acc2)
    outs = (o_pa, o_ap, o_pp)
    svs = (s_pa, s_ap, s_pp)
    tvs = (t_pa, t_ap, t_pp)
    htabs = (hauga, haugp, haugp)
    bufs = (buf0, buf1, buf2, buf3)
    gsems = (g0, g1, g2, g3)
    ssems = (s0, s1, s2, s3)
    row0 = sid * RPT

    for a in accs:
        pltpu.sync_copy(zrows, a.at[pl.ds(row0, RPT)])
    plsc.subcore_barrier()

    for et in range(3):
        pltpu.sync_copy(svs[et].at[wid], sref)
        pltpu.sync_copy(tvs[et].at[wid], tref)
        pltpu.sync_copy(x1h.at[et], x1tab)
        pltpu.sync_copy(h1h.at[et], h1tab)
        htab = htabs[et]
        acc = accs[et]

        pltpu.async_copy(htab.at[tref.at[0]], bufs[0], gsems[0])
        pltpu.async_copy(htab.at[tref.at[1]], bufs[1], gsems[1])

        def chunk4(i4, carry):
            for b in range(4):
                j = i4 * 4 + b
                rows = bufs[b]
                bp = (b + 2) % 4

                @pl.when(j >= 2)
                def _():
                    pltpu.make_async_copy(
                        bufs[bp], acc.at[sref.at[j - 2]], ssems[bp]).wait()

                @pl.when(j + 2 < NCH)
                def _():
                    pltpu.async_copy(htab.at[tref.at[j + 2]], bufs[bp],
                                     gsems[bp])

                pltpu.make_async_copy(htab.at[tref.at[j]], rows,
                                      gsems[b]).wait()

                def group(g, c2):
                    off = pl.multiple_of(g * 16, 16)
                    s16 = sref[j, pl.ds(off, 16)]
                    t16 = tref[j, pl.ds(off, 16)]
                    x1v = plsc.load_gather(x1tab, [s16])
                    h1v = plsc.load_gather(h1tab, [t16])
                    w1v = _exp_leaky(x1v + h1v)
                    ebase = g * 16
                    for k in range(16):
                        wb = jnp.take_along_axis(
                            w1v, jnp.full((16,), k, jnp.int32), axis=0)
                        e = ebase + k
                        for c in range(AUG // 16):
                            rows[e, pl.ds(c * 16, 16)] = (
                                rows[e, pl.ds(c * 16, 16)] * wb)
                    return c2

                lax.fori_loop(0, K // 16, group, 0)
                pltpu.async_copy(rows, acc.at[sref.at[j]], ssems[b], add=True)
            return carry

        lax.fori_loop(0, NCH // 4, chunk4, 0)
        for j in (NCH - 2, NCH - 1):
            b = j % 4
            pltpu.make_async_copy(bufs[b], acc.at[sref.at[j]],
                                  ssems[b]).wait()

    plsc.subcore_barrier()
    for et in range(3):
        pltpu.sync_copy(accs[et].at[pl.ds(row0, RPT)],
                        outs[et].at[cid, pl.ds(row0, RPT)])


@functools.cache
def _get_sc_edge():
  return pl.kernel(
    _sc_body,
    out_type=tuple(jax.ShapeDtypeStruct((NC, NPAD, AUG), jnp.float32)
                   for _ in range(3)),
    mesh=plsc.VectorSubcoreMesh(core_axis_name="c", subcore_axis_name="s",
                                num_cores=NC, num_subcores=NS),
    compiler_params=pltpu.CompilerParams(use_tc_tiling_on_sc=False,
                                         needs_layout_passes=False),
    scratch_types=(
        pltpu.VMEM_SHARED((NPAD, AUG), jnp.float32),
        pltpu.VMEM_SHARED((NPAD, AUG), jnp.float32),
        pltpu.VMEM_SHARED((NPAD, AUG), jnp.float32),
        pltpu.VMEM((NCH, K), jnp.int32),
        pltpu.VMEM((NCH, K), jnp.int32),
        pltpu.VMEM((NPAD,), jnp.float32),
        pltpu.VMEM((NPAD,), jnp.float32),
        pltpu.VMEM((K, AUG), jnp.float32),
        pltpu.VMEM((K, AUG), jnp.float32),
        pltpu.VMEM((K, AUG), jnp.float32),
        pltpu.VMEM((K, AUG), jnp.float32),
        pltpu.SemaphoreType.DMA,
        pltpu.SemaphoreType.DMA,
        pltpu.SemaphoreType.DMA,
        pltpu.SemaphoreType.DMA,
        pltpu.SemaphoreType.DMA,
        pltpu.SemaphoreType.DMA,
        pltpu.SemaphoreType.DMA,
        pltpu.SemaphoreType.DMA,
    ),
  )


def _sem_from_acc(acc_ref, xt, w2col):
    a = acc_ref[0, 0:N, 0:D] + acc_ref[1, 0:N, 0:D]
    d = acc_ref[0, 0:N, D:D + 1] + acc_ref[1, 0:N, D:D + 1]
    return (a + w2col * xt) / (d + w2col)


def _paper_update(z_pa, z_pp, sw1_ref, sb1_ref, sw2_ref):
    s1 = jnp.dot(jnp.tanh(jnp.dot(z_pa, sw1_ref[...],
                                  preferred_element_type=jnp.float32)
                          + sb1_ref[...]),
                 sw2_ref[...], preferred_element_type=jnp.float32)
    s2 = jnp.dot(jnp.tanh(jnp.dot(z_pp, sw1_ref[...],
                                  preferred_element_type=jnp.float32)
                          + sb1_ref[...]),
                 sw2_ref[...], preferred_element_type=jnp.float32)
    m1 = jnp.mean(s1)
    m2 = jnp.mean(s2)
    mx = jnp.maximum(m1, m2)
    e1 = jnp.exp(m1 - mx)
    e2 = jnp.exp(m2 - mx)
    tot = e1 + e2
    return _elu((e1 / tot) * z_pa + (e2 / tot) * z_pp)


def _combine_body(accpa_ref, accap_ref, accpp_ref, xtp_ref, xta_ref, w2_ref,
                  sw1_ref, sb1_ref, sw2_ref, a2n_ref,
                  haugp_ref, hauga_ref, h1n_ref):
    xtp = xtp_ref[...]
    xta = xta_ref[...]
    z_pa = _sem_from_acc(accpa_ref, xtp, w2_ref[:, 0:1])
    z_ap = _sem_from_acc(accap_ref, xta, w2_ref[:, 1:2])
    z_pp = _sem_from_acc(accpp_ref, xtp, w2_ref[:, 2:3])
    hp = _paper_update(z_pa, z_pp, sw1_ref, sb1_ref, sw2_ref)
    ha = _elu(z_ap)
    for ref, h in ((haugp_ref, hp), (hauga_ref, ha)):
        ref[...] = jnp.zeros((NPAD, AUG), jnp.float32)
        ref[0:N, 0:D] = h
        ref[0:N, D:D + 1] = jnp.ones((N, 1), jnp.float32)
    h1n_ref[...] = jnp.zeros((3, NPAD), jnp.float32)
    hts = (ha, hp, hp)
    for et in range(3):
        h1n_ref[et:et + 1, 0:N] = _rowmat(a2n_ref[et], hts[et])


_combine = pl.pallas_call(
    _combine_body,
    out_shape=(
        jax.ShapeDtypeStruct((NPAD, AUG), jnp.float32),
        jax.ShapeDtypeStruct((NPAD, AUG), jnp.float32),
        jax.ShapeDtypeStruct((3, NPAD), jnp.float32),
    ),
)


def _final_body(accpa_ref, accpp_ref, xtp_ref, w2_ref,
                sw1_ref, sb1_ref, sw2_ref, wfc2_ref, bfc2_ref, out_ref):
    xtp = xtp_ref[...]
    z_pa = _sem_from_acc(accpa_ref, xtp, w2_ref[:, 0:1])
    z_pp = _sem_from_acc(accpp_ref, xtp, w2_ref[:, 2:3])
    hp = _paper_update(z_pa, z_pp, sw1_ref, sb1_ref, sw2_ref)
    out_ref[...] = jnp.dot(hp, wfc2_ref[...],
                           preferred_element_type=jnp.float32) + bfc2_ref[...]


_final = pl.pallas_call(
    _final_body,
    out_shape=jax.ShapeDtypeStruct((N, 16), jnp.float32),
)


def kernel(x_paper, x_author, edge_index_pa, edge_index_ap, edge_index_pp,
           W_fc1_paper, b_fc1_paper, W_fc1_author, b_fc1_author,
           W_fc2, b_fc2, attn1, attn2, sem_W1, sem_b1, sem_W2):
    idx = []
    for ei in (edge_index_pa, edge_index_ap, edge_index_pp):
        ei = ei.astype(jnp.int32)
        s = jnp.concatenate([ei[0], jnp.full((EPAD - E,), N, jnp.int32)])
        t = jnp.concatenate([ei[1], jnp.zeros((EPAD - E,), jnp.int32)])
        idx.append((s.reshape(NW, NCH, K), t.reshape(NW, NCH, K)))
    zrows = jnp.zeros((RPT, AUG), jnp.float32)

    a1r = attn1.reshape(HOP, 3, 1, D)
    a2r = attn2.reshape(HOP, 3, 1, D)
    xtp, xta, haugp, hauga, x1_all, w2_all, h1 = _prologue(
        x_paper, x_author, W_fc1_paper, b_fc1_paper.reshape(1, D),
        W_fc1_author, b_fc1_author.reshape(1, D), a1r, a2r)

    out = None
    for i in range(HOP):
        acc_pa, acc_ap, acc_pp = _get_sc_edge()(
            idx[0][0], idx[0][1], idx[1][0], idx[1][1], idx[2][0], idx[2][1],
            x1_all[i], h1, haugp, hauga, zrows)
        sw1 = sem_W1[i, 0]
        sb1 = sem_b1[i, 0].reshape(1, -1)
        sw2 = sem_W2[i, 0].reshape(-1, 1)
        if i < HOP - 1:
            haugp, hauga, h1 = _combine(
                acc_pa, acc_ap, acc_pp, xtp, xta, w2_all[i],
                sw1, sb1, sw2, a2r[i + 1])
        else:
            out = _final(acc_pa, acc_pp, xtp, w2_all[i],
                         sw1, sb1, sw2, W_fc2, b_fc2.reshape(1, -1))
    return out


# speedup vs baseline: 12.4643x; 1.0658x over previous
import functools

import jax
import jax.numpy as jnp
from jax import lax
from jax.experimental import pallas as pl
from jax.experimental.pallas import tpu as pltpu
from jax.experimental.pallas import tpu_sc as plsc

N = 5000
NPAD = 5120
E = 160000
D = 64
AUG = 80
HOP = 5
NC = 2
NS = 16
NW = NC * NS
K = 64
NCH = 80
EPT = K * NCH
EPAD = EPT * NW
RPT = NPAD // NS


def _elu(x):
    return jnp.where(x > 0, x, jnp.exp(jnp.minimum(x, 0.0)) - 1.0)


def _exp_leaky(x):
    return jnp.exp(jnp.where(x >= 0, x, 0.2 * x))


def _rowmat(a_row, x):
    return lax.dot_general(a_row, x, (((1,), (1,)), ((), ())),
                           preferred_element_type=jnp.float32)


def _colmat(x, a_row):
    return lax.dot_general(x, a_row, (((1,), (1,)), ((), ())),
                           preferred_element_type=jnp.float32)


def _prologue_body(xp_ref, xa_ref, w1p_ref, b1p_ref, w1a_ref, b1a_ref,
                   a1_ref, a2_ref,
                   xtp_ref, xta_ref, haugp_ref, hauga_ref, x1_ref, w2_ref,
                   h10_ref):
    xtp = jnp.maximum(jnp.dot(xp_ref[...], w1p_ref[...],
                              preferred_element_type=jnp.float32)
                      + b1p_ref[...], 0.0)
    xta = jnp.maximum(jnp.dot(xa_ref[...], w1a_ref[...],
                              preferred_element_type=jnp.float32)
                      + b1a_ref[...], 0.0)
    xtp_ref[...] = xtp
    xta_ref[...] = xta

    for ref, xt in ((haugp_ref, xtp), (hauga_ref, xta)):
        ref[...] = jnp.zeros((NPAD, AUG), jnp.float32)
        ref[0:N, 0:D] = xt
        ref[0:N, D:D + 1] = jnp.ones((N, 1), jnp.float32)

    x1_ref[...] = jnp.zeros((HOP, 3, NPAD), jnp.float32)
    xts = (xtp, xta, xtp)
    for i in range(HOP):
        for et in range(3):
            x = xts[et]
            a1 = a1_ref[i, et]
            a2 = a2_ref[i, et]
            x1_ref[i, et:et + 1, 0:N] = _rowmat(a1, x)
            w2_ref[i, :, et:et + 1] = _exp_leaky(_colmat(x, a1)
                                                 + _colmat(x, a2))

    h10_ref[...] = jnp.zeros((3, NPAD), jnp.float32)
    hts = (xta, xtp, xtp)
    for et in range(3):
        h10_ref[et:et + 1, 0:N] = _rowmat(a2_ref[0, et], hts[et])


_prologue = pl.pallas_call(
    _prologue_body,
    out_shape=(
        jax.ShapeDtypeStruct((N, D), jnp.float32),
        jax.ShapeDtypeStruct((N, D), jnp.float32),
        jax.ShapeDtypeStruct((NPAD, AUG), jnp.float32),
        jax.ShapeDtypeStruct((NPAD, AUG), jnp.float32),
        jax.ShapeDtypeStruct((HOP, 3, NPAD), jnp.float32),
        jax.ShapeDtypeStruct((HOP, N, 3), jnp.float32),
        jax.ShapeDtypeStruct((3, NPAD), jnp.float32),
    ),
)


def _sc_body(s_pa, t_pa, s_ap, t_ap, s_pp, t_pp, x1h, h1h, haugp, hauga,
             zrows,
             o_pa, o_ap, o_pp,
             acc0, acc1, acc2, sref, tref, x1tab, h1tab,
             buf0, buf1, buf2, buf3, out0, out1, g0, g1, g2, g3, s0, s1):
    cid = lax.axis_index("c")
    sid = lax.axis_index("s")
    wid = sid * NC + cid
    accs = (acc0, acc1, acc2)
    outs = (o_pa, o_ap, o_pp)
    svs = (s_pa, s_ap, s_pp)
    tvs = (t_pa, t_ap, t_pp)
    htabs = (hauga, haugp, haugp)
    bufs = (buf0, buf1, buf2, buf3)
    routs = (out0, out1)
    gsems = (g0, g1, g2, g3)
    ssems = (s0, s1)
    row0 = sid * RPT

    for a in accs:
        pltpu.sync_copy(zrows, a.at[pl.ds(row0, RPT)])
    plsc.subcore_barrier()

    for et in range(3):
        pltpu.sync_copy(svs[et].at[wid], sref)
        pltpu.sync_copy(tvs[et].at[wid], tref)
        pltpu.sync_copy(x1h.at[et], x1tab)
        pltpu.sync_copy(h1h.at[et], h1tab)
        htab = htabs[et]
        acc = accs[et]

        pltpu.async_copy(htab.at[tref.at[0]], bufs[0], gsems[0])
        pltpu.async_copy(htab.at[tref.at[1]], bufs[1], gsems[1])

        def chunk4(i4, carry):
            for b in range(4):
                j = i4 * 4 + b
                rows = bufs[b]
                rout = routs[b % 2]
                ssem = ssems[b % 2]
                bp = (b + 2) % 4

                @pl.when(j >= 2)
                def _():
                    pltpu.make_async_copy(
                        rout, acc.at[sref.at[j - 2]], ssem).wait()

                @pl.when(j + 2 < NCH)
                def _():
                    pltpu.async_copy(htab.at[tref.at[j + 2]], bufs[bp],
                                     gsems[bp])

                pltpu.make_async_copy(htab.at[tref.at[j]], rows,
                                      gsems[b]).wait()

                def group(g, c2):
                    off = pl.multiple_of(g * 16, 16)
                    s16 = sref[j, pl.ds(off, 16)]
                    t16 = tref[j, pl.ds(off, 16)]
                    x1v = plsc.load_gather(x1tab, [s16])
                    h1v = plsc.load_gather(h1tab, [t16])
                    w1v = _exp_leaky(x1v + h1v)
                    nc = AUG // 16
                    for k4 in range(4):
                        ks = [k4 * 4 + k for k in range(4)]
                        wbs = [jnp.take_along_axis(
                            w1v, jnp.full((16,), k, jnp.int32), axis=0)
                            for k in ks]
                        es = [off + k for k in ks]
                        vals = [[rows[e, pl.ds(c * 16, 16)]
                                 for c in range(nc)] for e in es]
                        scl = [[vals[k][c] * wbs[k] for c in range(nc)]
                               for k in range(4)]
                        for k in range(4):
                            for c in range(nc):
                                rout[es[k], pl.ds(c * 16, 16)] = scl[k][c]
                    return c2

                lax.fori_loop(0, K // 16, group, 0)
                pltpu.async_copy(rout, acc.at[sref.at[j]], ssem, add=True)
            return carry

        lax.fori_loop(0, NCH // 4, chunk4, 0)
        for j in (NCH - 2, NCH - 1):
            pltpu.make_async_copy(routs[j % 2], acc.at[sref.at[j]],
                                  ssems[j % 2]).wait()

    plsc.subcore_barrier()
    for et in range(3):
        pltpu.sync_copy(accs[et].at[pl.ds(row0, RPT)],
                        outs[et].at[cid, pl.ds(row0, RPT)])


@functools.cache
def _get_sc_edge():
  return pl.kernel(
    _sc_body,
    out_type=tuple(jax.ShapeDtypeStruct((NC, NPAD, AUG), jnp.float32)
                   for _ in range(3)),
    mesh=plsc.VectorSubcoreMesh(core_axis_name="c", subcore_axis_name="s",
                                num_cores=NC, num_subcores=NS),
    compiler_params=pltpu.CompilerParams(use_tc_tiling_on_sc=False,
                                         needs_layout_passes=False),
    scratch_types=(
        pltpu.VMEM_SHARED((NPAD, AUG), jnp.float32),
        pltpu.VMEM_SHARED((NPAD, AUG), jnp.float32),
        pltpu.VMEM_SHARED((NPAD, AUG), jnp.float32),
        pltpu.VMEM((NCH, K), jnp.int32),
        pltpu.VMEM((NCH, K), jnp.int32),
        pltpu.VMEM((NPAD,), jnp.float32),
        pltpu.VMEM((NPAD,), jnp.float32),
        pltpu.VMEM((K, AUG), jnp.float32),
        pltpu.VMEM((K, AUG), jnp.float32),
        pltpu.VMEM((K, AUG), jnp.float32),
        pltpu.VMEM((K, AUG), jnp.float32),
        pltpu.VMEM((K, AUG), jnp.float32),
        pltpu.VMEM((K, AUG), jnp.float32),
        pltpu.SemaphoreType.DMA,
        pltpu.SemaphoreType.DMA,
        pltpu.SemaphoreType.DMA,
        pltpu.SemaphoreType.DMA,
        pltpu.SemaphoreType.DMA,
        pltpu.SemaphoreType.DMA,
    ),
  )


def _sem_from_acc(acc_ref, xt, w2col):
    a = acc_ref[0, 0:N, 0:D] + acc_ref[1, 0:N, 0:D]
    d = acc_ref[0, 0:N, D:D + 1] + acc_ref[1, 0:N, D:D + 1]
    return (a + w2col * xt) / (d + w2col)


def _paper_update(z_pa, z_pp, sw1_ref, sb1_ref, sw2_ref):
    s1 = jnp.dot(jnp.tanh(jnp.dot(z_pa, sw1_ref[...],
                                  preferred_element_type=jnp.float32)
                          + sb1_ref[...]),
                 sw2_ref[...], preferred_element_type=jnp.float32)
    s2 = jnp.dot(jnp.tanh(jnp.dot(z_pp, sw1_ref[...],
                                  preferred_element_type=jnp.float32)
                          + sb1_ref[...]),
                 sw2_ref[...], preferred_element_type=jnp.float32)
    m1 = jnp.mean(s1)
    m2 = jnp.mean(s2)
    mx = jnp.maximum(m1, m2)
    e1 = jnp.exp(m1 - mx)
    e2 = jnp.exp(m2 - mx)
    tot = e1 + e2
    return _elu((e1 / tot) * z_pa + (e2 / tot) * z_pp)


def _combine_body(accpa_ref, accap_ref, accpp_ref, xtp_ref, xta_ref, w2_ref,
                  sw1_ref, sb1_ref, sw2_ref, a2n_ref,
                  haugp_ref, hauga_ref, h1n_ref):
    xtp = xtp_ref[...]
    xta = xta_ref[...]
    z_pa = _sem_from_acc(accpa_ref, xtp, w2_ref[:, 0:1])
    z_ap = _sem_from_acc(accap_ref, xta, w2_ref[:, 1:2])
    z_pp = _sem_from_acc(accpp_ref, xtp, w2_ref[:, 2:3])
    hp = _paper_update(z_pa, z_pp, sw1_ref, sb1_ref, sw2_ref)
    ha = _elu(z_ap)
    for ref, h in ((haugp_ref, hp), (hauga_ref, ha)):
        ref[...] = jnp.zeros((NPAD, AUG), jnp.float32)
        ref[0:N, 0:D] = h
        ref[0:N, D:D + 1] = jnp.ones((N, 1), jnp.float32)
    h1n_ref[...] = jnp.zeros((3, NPAD), jnp.float32)
    hts = (ha, hp, hp)
    for et in range(3):
        h1n_ref[et:et + 1, 0:N] = _rowmat(a2n_ref[et], hts[et])


_combine = pl.pallas_call(
    _combine_body,
    out_shape=(
        jax.ShapeDtypeStruct((NPAD, AUG), jnp.float32),
        jax.ShapeDtypeStruct((NPAD, AUG), jnp.float32),
        jax.ShapeDtypeStruct((3, NPAD), jnp.float32),
    ),
)


def _final_body(accpa_ref, accpp_ref, xtp_ref, w2_ref,
                sw1_ref, sb1_ref, sw2_ref, wfc2_ref, bfc2_ref, out_ref):
    xtp = xtp_ref[...]
    z_pa = _sem_from_acc(accpa_ref, xtp, w2_ref[:, 0:1])
    z_pp = _sem_from_acc(accpp_ref, xtp, w2_ref[:, 2:3])
    hp = _paper_update(z_pa, z_pp, sw1_ref, sb1_ref, sw2_ref)
    out_ref[...] = jnp.dot(hp, wfc2_ref[...],
                           preferred_element_type=jnp.float32) + bfc2_ref[...]


_final = pl.pallas_call(
    _final_body,
    out_shape=jax.ShapeDtypeStruct((N, 16), jnp.float32),
)


def kernel(x_paper, x_author, edge_index_pa, edge_index_ap, edge_index_pp,
           W_fc1_paper, b_fc1_paper, W_fc1_author, b_fc1_author,
           W_fc2, b_fc2, attn1, attn2, sem_W1, sem_b1, sem_W2):
    idx = []
    for ei in (edge_index_pa, edge_index_ap, edge_index_pp):
        ei = ei.astype(jnp.int32)
        s = jnp.concatenate([ei[0], jnp.full((EPAD - E,), N, jnp.int32)])
        t = jnp.concatenate([ei[1], jnp.zeros((EPAD - E,), jnp.int32)])
        idx.append((s.reshape(NW, NCH, K), t.reshape(NW, NCH, K)))
    zrows = jnp.zeros((RPT, AUG), jnp.float32)

    a1r = attn1.reshape(HOP, 3, 1, D)
    a2r = attn2.reshape(HOP, 3, 1, D)
    xtp, xta, haugp, hauga, x1_all, w2_all, h1 = _prologue(
        x_paper, x_author, W_fc1_paper, b_fc1_paper.reshape(1, D),
        W_fc1_author, b_fc1_author.reshape(1, D), a1r, a2r)

    out = None
    for i in range(HOP):
        acc_pa, acc_ap, acc_pp = _get_sc_edge()(
            idx[0][0], idx[0][1], idx[1][0], idx[1][1], idx[2][0], idx[2][1],
            x1_all[i], h1, haugp, hauga, zrows)
        sw1 = sem_W1[i, 0]
        sb1 = sem_b1[i, 0].reshape(1, -1)
        sw2 = sem_W2[i, 0].reshape(-1, 1)
        if i < HOP - 1:
            haugp, hauga, h1 = _combine(
                acc_pa, acc_ap, acc_pp, xtp, xta, w2_all[i],
                sw1, sb1, sw2, a2r[i + 1])
        else:
            out = _final(acc_pa, acc_pp, xtp, w2_all[i],
                         sw1, sb1, sw2, W_fc2, b_fc2.reshape(1, -1))
    return out
